# baseline (speedup 1.0000x reference)
import functools

import jax
import jax.numpy as jnp
from jax import lax
from jax.experimental import pallas as pl
from jax.experimental.pallas import tpu as pltpu
from jax.experimental.pallas import tpu_sc as plsc

WIDTH = 32
CH = 128
NC = 2
NS = 16
NW = NC * NS


def _mesh():
    return plsc.VectorSubcoreMesh(core_axis_name="c", subcore_axis_name="s")


def _worker_id():
    return lax.axis_index("s") * NC + lax.axis_index("c")


def _chunk_split(nchunks):
    base = nchunks // NW
    extra = nchunks % NW
    return base, extra


def _make_gather(n, e):
    nchunks = e // CH
    base_c, extra_c = _chunk_split(nchunks)

    @functools.partial(
        pl.kernel,
        mesh=_mesh(),
        out_type=jax.ShapeDtypeStruct((e, WIDTH), jnp.float32),
        compiler_params=pltpu.CompilerParams(use_tc_tiling_on_sc=False),
        scratch_types=[
            pltpu.VMEM((CH,), jnp.int32),
            pltpu.VMEM((CH, WIDTH), jnp.float32),
            pltpu.SemaphoreType.DMA,
        ],
    )
    def gather(h_hbm, src_hbm, out_hbm, idx_v, rows_v, sem):
        wid = _worker_id()
        nloc = base_c + (wid < extra_c).astype(jnp.int32)
        start = wid * base_c + jnp.minimum(wid, extra_c)

        def body(i, carry):
            off = (start + i) * CH
            pltpu.sync_copy(src_hbm.at[pl.ds(off, CH)], idx_v)
            pltpu.async_copy(h_hbm.at[idx_v], rows_v, sem).wait()
            pltpu.sync_copy(rows_v, out_hbm.at[pl.ds(off, CH)])
            return carry

        lax.fori_loop(0, nloc, body, 0)

    return gather


def _make_scatter(n, e, width):
    nchunks = e // CH
    base_c, extra_c = _chunk_split(nchunks)
    rows_per_tile = n // NS

    @functools.partial(
        pl.kernel,
        mesh=_mesh(),
        out_type=jax.ShapeDtypeStruct((NC * n, width), jnp.float32),
        compiler_params=pltpu.CompilerParams(use_tc_tiling_on_sc=False),
        scratch_types=[
            pltpu.VMEM_SHARED((n, width), jnp.float32),
            pltpu.VMEM((CH,), jnp.int32),
            pltpu.VMEM((CH, width), jnp.float32),
            pltpu.VMEM((rows_per_tile, width), jnp.float32),
            pltpu.SemaphoreType.DMA,
        ],
    )
    def scatter(rows_hbm, dst_hbm, zeros_hbm, out_hbm, acc_sh, idx_v, rows_v,
                stage_v, sem):
        cid = lax.axis_index("c")
        sid = lax.axis_index("s")
        wid = sid * NC + cid
        r0 = sid * rows_per_tile

        pltpu.sync_copy(zeros_hbm.at[pl.ds(r0, rows_per_tile)], stage_v)
        pltpu.sync_copy(stage_v, acc_sh.at[pl.ds(r0, rows_per_tile)])
        plsc.subcore_barrier()

        nloc = base_c + (wid < extra_c).astype(jnp.int32)
        start = wid * base_c + jnp.minimum(wid, extra_c)

        def body(i, carry):
            off = (start + i) * CH
            pltpu.sync_copy(dst_hbm.at[pl.ds(off, CH)], idx_v)
            pltpu.sync_copy(rows_hbm.at[pl.ds(off, CH)], rows_v)
            pltpu.sync_copy(rows_v, acc_sh.at[idx_v], add=True)
            return carry

        lax.fori_loop(0, nloc, body, 0)
        plsc.subcore_barrier()

        pltpu.sync_copy(acc_sh.at[pl.ds(r0, rows_per_tile)], stage_v)
        pltpu.sync_copy(stage_v, out_hbm.at[pl.ds(cid * n + r0, rows_per_tile)])

    return scatter


def _lin_body(x_ref, w_ref, b_ref, o_ref, *, relu):
    y = jnp.dot(x_ref[...], w_ref[...], preferred_element_type=jnp.float32)
    y = y + b_ref[...]
    o_ref[...] = jnp.maximum(y, 0.0) if relu else y


def _tc_linear(x, w, b, relu=False):
    n, _ = x.shape
    fo = w.shape[1]
    return pl.pallas_call(
        functools.partial(_lin_body, relu=relu),
        out_shape=jax.ShapeDtypeStruct((n, fo), jnp.float32),
    )(x, w, b.reshape(1, fo))


def _msg_body(ea_ref, hs_ref, k1w, k1b, k2w, k2b, k3w, k3b, o_ref, *, eb):
    a = jnp.dot(ea_ref[...], k1w[...], preferred_element_type=jnp.float32)
    a = jnp.maximum(a + k1b[...], 0.0)
    a = jnp.dot(a, k2w[...], preferred_element_type=jnp.float32)
    a = jnp.maximum(a + k2b[...], 0.0)
    w = jnp.dot(a, k3w[...], preferred_element_type=jnp.float32) + k3b[...]
    h = hs_ref[...]
    acc = h[:, 0:1] * w[:, 0:WIDTH]
    for i in range(1, WIDTH):
        acc = acc + h[:, i:i + 1] * w[:, i * WIDTH:(i + 1) * WIDTH]
    o_ref[...] = acc


def _tc_msg(edge_attr, h_src, k1_w, k1_b, k2_w, k2_b, k3_w, k3_b, eb=640):
    e, ki = edge_attr.shape
    kw2 = k2_w.shape[1]
    kw1 = k1_w.shape[1]
    grid = e // eb
    full = lambda i: (0, 0)
    return pl.pallas_call(
        functools.partial(_msg_body, eb=eb),
        grid=(grid,),
        in_specs=[
            pl.BlockSpec((eb, ki), lambda i: (i, 0)),
            pl.BlockSpec((eb, WIDTH), lambda i: (i, 0)),
            pl.BlockSpec(k1_w.shape, full),
            pl.BlockSpec((1, kw1), full),
            pl.BlockSpec(k2_w.shape, full),
            pl.BlockSpec((1, kw2), full),
            pl.BlockSpec(k3_w.shape, full),
            pl.BlockSpec((1, WIDTH * WIDTH), full),
        ],
        out_specs=pl.BlockSpec((eb, WIDTH), lambda i: (i, 0)),
        out_shape=jax.ShapeDtypeStruct((e, WIDTH), jnp.float32),
    )(edge_attr, h_src, k1_w, k1_b.reshape(1, kw1), k2_w, k2_b.reshape(1, kw2),
      k3_w, k3_b.reshape(1, WIDTH * WIDTH))


def _update_body(p_ref, c_ref, h_ref, rw_ref, cb_ref, o_ref, *, n, relu):
    cnt = jnp.maximum(c_ref[0:n, 0:1] + c_ref[n:2 * n, 0:1], 1.0)
    agg = (p_ref[0:n, :] + p_ref[n:2 * n, :]) / cnt
    y = agg + jnp.dot(h_ref[...], rw_ref[...],
                      preferred_element_type=jnp.float32) + cb_ref[...]
    o_ref[...] = jnp.maximum(y, 0.0) if relu else y


def _tc_update(parts, cnts, h, root_w, conv_b, relu):
    n = h.shape[0]
    return pl.pallas_call(
        functools.partial(_update_body, n=n, relu=relu),
        out_shape=jax.ShapeDtypeStruct((n, WIDTH), jnp.float32),
    )(parts, cnts, h, root_w, conv_b.reshape(1, WIDTH))


def _head_body(h_ref, w2_ref, b2_ref, w3_ref, b3_ref, o_ref):
    a = jnp.dot(h_ref[...], w2_ref[...], preferred_element_type=jnp.float32)
    a = jnp.maximum(a + b2_ref[...], 0.0)
    o_ref[...] = jnp.dot(a, w3_ref[...],
                         preferred_element_type=jnp.float32) + b3_ref[...]


def _tc_head(h, fc2_w, fc2_b, fc3_w, fc3_b):
    n = h.shape[0]
    kw = fc2_w.shape[1]
    return pl.pallas_call(
        _head_body,
        out_shape=jax.ShapeDtypeStruct((n, 1), jnp.float32),
    )(h, fc2_w, fc2_b.reshape(1, kw), fc3_w, fc3_b.reshape(1, 1))


def kernel(x, edge_index, edge_attr, fc1_w, fc1_b, k1_w, k1_b, k2_w, k2_b,
           k3_w, k3_b, root_w, conv_b, fc2_w, fc2_b, fc3_w, fc3_b):
    n = x.shape[0]
    e = edge_attr.shape[0]
    src = edge_index[0].astype(jnp.int32)
    dst = edge_index[1].astype(jnp.int32)

    gather = _make_gather(n, e)
    scatter32 = _make_scatter(n, e, WIDTH)
    scatter16 = _make_scatter(n, e, 16)

    zeros32 = jnp.zeros((n, WIDTH), jnp.float32)
    zeros16 = jnp.zeros((n, 16), jnp.float32)
    ones16 = jnp.ones((e, 16), jnp.float32)

    h = _tc_linear(x, fc1_w, fc1_b)
    cnts = scatter16(ones16, dst, zeros16)

    depth = 2
    for k in range(depth):
        h_src = gather(h, src)
        msg = _tc_msg(edge_attr, h_src, k1_w, k1_b, k2_w, k2_b, k3_w, k3_b)
        parts = scatter32(msg, dst, zeros32)
        h = _tc_update(parts, cnts, h, root_w, conv_b, relu=(k != depth - 1))

    return _tc_head(h, fc2_w, fc2_b, fc3_w, fc3_b)


# speedup vs baseline: 2.3853x; 2.3853x over previous
import functools

import jax
import jax.numpy as jnp
from jax import lax
from jax.experimental import pallas as pl
from jax.experimental.pallas import tpu as pltpu
from jax.experimental.pallas import tpu_sc as plsc

WIDTH = 32
CH = 128
NC = 2
NS = 16
NW = NC * NS


def _mesh():
    return plsc.VectorSubcoreMesh(core_axis_name="c", subcore_axis_name="s")


def _worker_id():
    return lax.axis_index("s") * NC + lax.axis_index("c")


def _chunk_split(nchunks):
    base = nchunks // NW
    extra = nchunks % NW
    return base, extra


def _make_gather(n, e):
    nchunks = e // CH
    base_c, extra_c = _chunk_split(nchunks)

    @functools.partial(
        pl.kernel,
        mesh=_mesh(),
        out_type=jax.ShapeDtypeStruct((e, WIDTH), jnp.float32),
        compiler_params=pltpu.CompilerParams(use_tc_tiling_on_sc=False),
        scratch_types=[
            pltpu.VMEM((CH,), jnp.int32),
            pltpu.VMEM((CH, WIDTH), jnp.float32),
            pltpu.SemaphoreType.DMA,
        ],
    )
    def gather(h_hbm, src_hbm, out_hbm, idx_v, rows_v, sem):
        wid = _worker_id()
        nloc = base_c + (wid < extra_c).astype(jnp.int32)
        start = wid * base_c + jnp.minimum(wid, extra_c)

        def body(i, carry):
            off = (start + i) * CH
            pltpu.sync_copy(src_hbm.at[pl.ds(off, CH)], idx_v)
            pltpu.async_copy(h_hbm.at[idx_v], rows_v, sem).wait()
            pltpu.sync_copy(rows_v, out_hbm.at[pl.ds(off, CH)])
            return carry

        lax.fori_loop(0, nloc, body, 0)

    return gather


def _make_scatter(n, e, width):
    nchunks = e // CH
    base_c, extra_c = _chunk_split(nchunks)
    rows_per_tile = n // NS

    @functools.partial(
        pl.kernel,
        mesh=_mesh(),
        out_type=jax.ShapeDtypeStruct((NC * n, width), jnp.float32),
        compiler_params=pltpu.CompilerParams(use_tc_tiling_on_sc=False),
        scratch_types=[
            pltpu.VMEM_SHARED((n, width), jnp.float32),
            pltpu.VMEM((CH,), jnp.int32),
            pltpu.VMEM((CH, width), jnp.float32),
            pltpu.VMEM((rows_per_tile, width), jnp.float32),
            pltpu.SemaphoreType.DMA,
        ],
    )
    def scatter(rows_hbm, dst_hbm, zeros_hbm, out_hbm, acc_sh, idx_v, rows_v,
                stage_v, sem):
        cid = lax.axis_index("c")
        sid = lax.axis_index("s")
        wid = sid * NC + cid
        r0 = sid * rows_per_tile

        pltpu.sync_copy(zeros_hbm.at[pl.ds(r0, rows_per_tile)], stage_v)
        pltpu.sync_copy(stage_v, acc_sh.at[pl.ds(r0, rows_per_tile)])
        plsc.subcore_barrier()

        nloc = base_c + (wid < extra_c).astype(jnp.int32)
        start = wid * base_c + jnp.minimum(wid, extra_c)

        def body(i, carry):
            off = (start + i) * CH
            pltpu.sync_copy(dst_hbm.at[pl.ds(off, CH)], idx_v)
            pltpu.sync_copy(rows_hbm.at[pl.ds(off, CH)], rows_v)
            pltpu.sync_copy(rows_v, acc_sh.at[idx_v], add=True)
            return carry

        lax.fori_loop(0, nloc, body, 0)
        plsc.subcore_barrier()

        pltpu.sync_copy(acc_sh.at[pl.ds(r0, rows_per_tile)], stage_v)
        pltpu.sync_copy(stage_v, out_hbm.at[pl.ds(cid * n + r0, rows_per_tile)])

    return scatter


def _lin_body(x_ref, w_ref, b_ref, o_ref, *, relu):
    y = jnp.dot(x_ref[...], w_ref[...], preferred_element_type=jnp.float32)
    y = y + b_ref[...]
    o_ref[...] = jnp.maximum(y, 0.0) if relu else y


def _tc_linear(x, w, b, relu=False):
    n, _ = x.shape
    fo = w.shape[1]
    return pl.pallas_call(
        functools.partial(_lin_body, relu=relu),
        out_shape=jax.ShapeDtypeStruct((n, fo), jnp.float32),
    )(x, w, b.reshape(1, fo))


def _msg_body(ea_ref, hs_ref, k1w, k1b, k2w, k2b, k3w, rep_ref, sel_ref,
              kb_ref, o_ref):
    a = jnp.dot(ea_ref[...], k1w[...], preferred_element_type=jnp.float32)
    a = jnp.maximum(a + k1b[...], 0.0)
    a = jnp.dot(a, k2w[...], preferred_element_type=jnp.float32)
    a = jnp.maximum(a + k2b[...], 0.0)
    w = jnp.dot(a, k3w[...], preferred_element_type=jnp.float32)
    h = hs_ref[...]
    hrep = jnp.dot(h, rep_ref[...], preferred_element_type=jnp.float32)
    msg = jnp.dot(hrep * w, sel_ref[...], preferred_element_type=jnp.float32)
    o_ref[...] = msg + jnp.dot(h, kb_ref[...],
                               preferred_element_type=jnp.float32)


def _tc_msg(edge_attr, h_src, k1_w, k1_b, k2_w, k2_b, k3_w, k3_b, eb=640):
    e, ki = edge_attr.shape
    kw2 = k2_w.shape[1]
    kw1 = k1_w.shape[1]
    ww = WIDTH * WIDTH
    grid = e // eb
    full = lambda i: (0, 0)
    j = jnp.arange(ww)
    rep = (j[None, :] // WIDTH == jnp.arange(WIDTH)[:, None]).astype(jnp.float32)
    sel = (j[:, None] % WIDTH == jnp.arange(WIDTH)[None, :]).astype(jnp.float32)
    return pl.pallas_call(
        _msg_body,
        grid=(grid,),
        in_specs=[
            pl.BlockSpec((eb, ki), lambda i: (i, 0)),
            pl.BlockSpec((eb, WIDTH), lambda i: (i, 0)),
            pl.BlockSpec(k1_w.shape, full),
            pl.BlockSpec((1, kw1), full),
            pl.BlockSpec(k2_w.shape, full),
            pl.BlockSpec((1, kw2), full),
            pl.BlockSpec(k3_w.shape, full),
            pl.BlockSpec((WIDTH, ww), full),
            pl.BlockSpec((ww, WIDTH), full),
            pl.BlockSpec((WIDTH, WIDTH), full),
        ],
        out_specs=pl.BlockSpec((eb, WIDTH), lambda i: (i, 0)),
        out_shape=jax.ShapeDtypeStruct((e, WIDTH), jnp.float32),
    )(edge_attr, h_src, k1_w, k1_b.reshape(1, kw1), k2_w, k2_b.reshape(1, kw2),
      k3_w, rep, sel, k3_b.reshape(WIDTH, WIDTH))


def _update_body(p_ref, c_ref, h_ref, rw_ref, cb_ref, o_ref, *, n, relu):
    cnt = jnp.maximum(c_ref[0:n, 0:1] + c_ref[n:2 * n, 0:1], 1.0)
    agg = (p_ref[0:n, :] + p_ref[n:2 * n, :]) / cnt
    y = agg + jnp.dot(h_ref[...], rw_ref[...],
                      preferred_element_type=jnp.float32) + cb_ref[...]
    o_ref[...] = jnp.maximum(y, 0.0) if relu else y


def _tc_update(parts, cnts, h, root_w, conv_b, relu):
    n = h.shape[0]
    return pl.pallas_call(
        functools.partial(_update_body, n=n, relu=relu),
        out_shape=jax.ShapeDtypeStruct((n, WIDTH), jnp.float32),
    )(parts, cnts, h, root_w, conv_b.reshape(1, WIDTH))


def _head_body(h_ref, w2_ref, b2_ref, w3_ref, b3_ref, o_ref):
    a = jnp.dot(h_ref[...], w2_ref[...], preferred_element_type=jnp.float32)
    a = jnp.maximum(a + b2_ref[...], 0.0)
    o_ref[...] = jnp.dot(a, w3_ref[...],
                         preferred_element_type=jnp.float32) + b3_ref[...]


def _tc_head(h, fc2_w, fc2_b, fc3_w, fc3_b):
    n = h.shape[0]
    kw = fc2_w.shape[1]
    return pl.pallas_call(
        _head_body,
        out_shape=jax.ShapeDtypeStruct((n, 1), jnp.float32),
    )(h, fc2_w, fc2_b.reshape(1, kw), fc3_w, fc3_b.reshape(1, 1))


def kernel(x, edge_index, edge_attr, fc1_w, fc1_b, k1_w, k1_b, k2_w, k2_b,
           k3_w, k3_b, root_w, conv_b, fc2_w, fc2_b, fc3_w, fc3_b):
    n = x.shape[0]
    e = edge_attr.shape[0]
    src = edge_index[0].astype(jnp.int32)
    dst = edge_index[1].astype(jnp.int32)

    gather = _make_gather(n, e)
    scatter32 = _make_scatter(n, e, WIDTH)
    scatter16 = _make_scatter(n, e, 16)

    zeros32 = jnp.zeros((n, WIDTH), jnp.float32)
    zeros16 = jnp.zeros((n, 16), jnp.float32)
    ones16 = jnp.ones((e, 16), jnp.float32)

    h = _tc_linear(x, fc1_w, fc1_b)
    cnts = scatter16(ones16, dst, zeros16)

    depth = 2
    for k in range(depth):
        h_src = gather(h, src)
        msg = _tc_msg(edge_attr, h_src, k1_w, k1_b, k2_w, k2_b, k3_w, k3_b)
        parts = scatter32(msg, dst, zeros32)
        h = _tc_update(parts, cnts, h, root_w, conv_b, relu=(k != depth - 1))

    return _tc_head(h, fc2_w, fc2_b, fc3_w, fc3_b)
